# Initial kernel scaffold; baseline (speedup 1.0000x reference)
import jax
import jax.numpy as jnp
from jax.experimental import pallas as pl


def kernel(x, weight):
    raise NotImplementedError("write your pallas kernel here")


# speedup vs baseline: 1.4357x; 1.4357x over previous
import functools

import jax
import jax.numpy as jnp
from jax import lax
from jax.experimental import pallas as pl
from jax.experimental.pallas import tpu as pltpu
from jax.experimental.pallas import tpu_sc as plsc

_NUM_WORKERS = 32
_CHUNK = 128


@functools.lru_cache(maxsize=None)
def _build_gather(total_rows: int, embed_dim: int):
    assert total_rows % (_NUM_WORKERS * _CHUNK) == 0
    rows_per_w = total_rows // _NUM_WORKERS
    steps = rows_per_w // _CHUNK
    mesh = plsc.VectorSubcoreMesh(core_axis_name="c", subcore_axis_name="s")

    @functools.partial(
        pl.kernel,
        out_type=jax.ShapeDtypeStruct((total_rows, embed_dim), jnp.float32),
        mesh=mesh,
        scratch_types=[
            pltpu.VMEM((steps, _CHUNK), jnp.int32),
            pltpu.VMEM((_CHUNK, embed_dim), jnp.float32),
            pltpu.SemaphoreType.DMA,
        ],
        compiler_params=pltpu.CompilerParams(use_tc_tiling_on_sc=False),
    )
    def gather_kernel(table_hbm, idx_hbm, out_hbm, idx_v, buf, sem):
        num_cores = lax.axis_size("c")
        wid = lax.axis_index("s") * num_cores + lax.axis_index("c")
        base = wid * rows_per_w
        pltpu.sync_copy(idx_hbm.at[pl.ds(wid * steps, steps)], idx_v)

        def body(j, carry):
            pltpu.async_copy(table_hbm.at[idx_v.at[j]], buf, sem).wait()
            pltpu.sync_copy(buf, out_hbm.at[pl.ds(base + j * _CHUNK, _CHUNK)])
            return carry

        lax.fori_loop(0, steps, body, 0)

    return gather_kernel


def kernel(x, weight):
    batch, num_fields = x.shape
    _, embed_dim = weight.shape
    total = batch * num_fields
    idx = x.reshape(total // _CHUNK, _CHUNK).astype(jnp.int32)
    out = _build_gather(total, embed_dim)(weight, idx)
    return out.reshape(batch, num_fields, embed_dim)


# speedup vs baseline: 1.5749x; 1.0969x over previous
import functools

import jax
import jax.numpy as jnp
from jax import lax
from jax.experimental import pallas as pl
from jax.experimental.pallas import tpu as pltpu
from jax.experimental.pallas import tpu_sc as plsc

_NUM_WORKERS = 32
_CHUNK = 128


@functools.lru_cache(maxsize=None)
def _build_gather(total_rows: int, embed_dim: int):
    assert total_rows % (_NUM_WORKERS * _CHUNK) == 0
    rows_per_w = total_rows // _NUM_WORKERS
    steps = rows_per_w // _CHUNK
    mesh = plsc.VectorSubcoreMesh(core_axis_name="c", subcore_axis_name="s")

    group = 4
    grows = group * _CHUNK
    ngroups = steps // group
    assert steps % group == 0 and ngroups % 2 == 0

    @functools.partial(
        pl.kernel,
        out_type=jax.ShapeDtypeStruct((total_rows, embed_dim), jnp.float32),
        mesh=mesh,
        scratch_types=[
            pltpu.VMEM((steps, _CHUNK), jnp.int32),
            pltpu.VMEM((grows, embed_dim), jnp.float32),
            pltpu.VMEM((grows, embed_dim), jnp.float32),
            pltpu.SemaphoreType.DMA,
            pltpu.SemaphoreType.DMA,
        ],
        compiler_params=pltpu.CompilerParams(use_tc_tiling_on_sc=False),
    )
    def gather_kernel(table_hbm, idx_hbm, out_hbm, idx_v, buf_a, buf_b, sem_a, sem_b):
        num_cores = lax.axis_size("c")
        wid = lax.axis_index("s") * num_cores + lax.axis_index("c")
        base = wid * rows_per_w
        pltpu.sync_copy(idx_hbm.at[pl.ds(wid * steps, steps)], idx_v)

        def fire(g, buf, sem):
            for k in range(group):
                pltpu.async_copy(
                    table_hbm.at[idx_v.at[g * group + k]],
                    buf.at[pl.ds(k * _CHUNK, _CHUNK)],
                    sem,
                )

        def drain_write(g, buf, sem):
            pltpu.make_async_copy(table_hbm.at[pl.ds(0, grows)], buf, sem).wait()
            pltpu.sync_copy(buf, out_hbm.at[pl.ds(base + g * grows, grows)])

        fire(0, buf_a, sem_a)

        def body(p, carry):
            g = 2 * p
            fire(g + 1, buf_b, sem_b)
            drain_write(g, buf_a, sem_a)

            @pl.when(p + 1 < ngroups // 2)
            def _():
                fire(g + 2, buf_a, sem_a)

            drain_write(g + 1, buf_b, sem_b)
            return carry

        lax.fori_loop(0, ngroups // 2, body, 0)

    return gather_kernel


def kernel(x, weight):
    batch, num_fields = x.shape
    _, embed_dim = weight.shape
    total = batch * num_fields
    idx = x.reshape(total // _CHUNK, _CHUNK).astype(jnp.int32)
    out = _build_gather(total, embed_dim)(weight, idx)
    return out.reshape(batch, num_fields, embed_dim)


# speedup vs baseline: 1.6701x; 1.0605x over previous
import functools

import jax
import jax.numpy as jnp
from jax import lax
from jax.experimental import pallas as pl
from jax.experimental.pallas import tpu as pltpu
from jax.experimental.pallas import tpu_sc as plsc

_NUM_WORKERS = 32
_CHUNK = 128
_GROUP = 4


@functools.lru_cache(maxsize=None)
def _build_gather(batch: int, num_fields: int, embed_dim: int):
    assert batch % (_NUM_WORKERS * _CHUNK) == 0
    cols_per_w = batch // _NUM_WORKERS
    chunks_per_f = cols_per_w // _CHUNK
    steps = num_fields * chunks_per_f
    assert steps % (2 * _GROUP) == 0
    grows = _GROUP * _CHUNK
    mesh = plsc.VectorSubcoreMesh(core_axis_name="c", subcore_axis_name="s")

    @functools.partial(
        pl.kernel,
        out_type=jax.ShapeDtypeStruct((num_fields, batch, embed_dim), jnp.float32),
        mesh=mesh,
        scratch_types=[
            pltpu.VMEM((num_fields, cols_per_w), jnp.int32),
            pltpu.VMEM((grows, embed_dim), jnp.float32),
            pltpu.VMEM((grows, embed_dim), jnp.float32),
            pltpu.SemaphoreType.DMA,
            pltpu.SemaphoreType.DMA,
        ],
        compiler_params=pltpu.CompilerParams(use_tc_tiling_on_sc=False),
    )
    def gather_kernel(table_hbm, xt_hbm, out_hbm, idx_v, buf_a, buf_b, sem_a, sem_b):
        num_cores = lax.axis_size("c")
        wid = lax.axis_index("s") * num_cores + lax.axis_index("c")
        base = wid * cols_per_w
        pltpu.sync_copy(xt_hbm.at[:, pl.ds(base, cols_per_w)], idx_v)

        def fire(g, buf, sem):
            for k in range(_GROUP):
                s = g * _GROUP + k
                f = s // chunks_per_f
                c = s % chunks_per_f
                pltpu.async_copy(
                    table_hbm.at[idx_v.at[f, pl.ds(c * _CHUNK, _CHUNK)]],
                    buf.at[pl.ds(k * _CHUNK, _CHUNK)],
                    sem,
                )

        def drain_write(g, buf, sem):
            pltpu.make_async_copy(table_hbm.at[pl.ds(0, grows)], buf, sem).wait()
            f = (g * _GROUP) // chunks_per_f
            c = (g * _GROUP) % chunks_per_f
            pltpu.sync_copy(buf, out_hbm.at[f, pl.ds(base + c * _CHUNK, grows)])

        assert chunks_per_f % _GROUP == 0

        fire(0, buf_a, sem_a)

        def body(p, carry):
            g = 2 * p
            fire(g + 1, buf_b, sem_b)
            drain_write(g, buf_a, sem_a)

            @pl.when(p + 1 < steps // (2 * _GROUP))
            def _():
                fire(g + 2, buf_a, sem_a)

            drain_write(g + 1, buf_b, sem_b)
            return carry

        lax.fori_loop(0, steps // (2 * _GROUP), body, 0)

    return gather_kernel


def kernel(x, weight):
    batch, num_fields = x.shape
    _, embed_dim = weight.shape
    out_t = _build_gather(batch, num_fields, embed_dim)(
        weight, x.T.astype(jnp.int32)
    )
    return out_t.transpose(1, 0, 2)
